# baseline (speedup 1.0000x reference)
import jax
import jax.numpy as jnp
from jax import lax
from jax.experimental import pallas as pl
from jax.experimental.pallas import tpu as pltpu
from jax.experimental.pallas import tpu_sc as plsc

BATCH = 4096
MAX_LEN = 200
WORD_DIM = 64
POS_DIM = 16
OUT_DIM = WORD_DIM + POS_DIM
NW = 32
BTILE = BATCH // NW
NSET = 5
NITER = MAX_LEN // NSET


def _out_slab(out_hbm, t, wid):
    return out_hbm.at[pl.ds(t * 10, 10), pl.ds(wid, 1), :]


def _sc_body(wt_hbm, pt_hbm, ww_hbm, wp_hbm, out_hbm,
             iw_all, ip_all,
             rw0, rp0, rw1, rp1, rw2, rp2, rw3, rp3, rw4, rp4,
             tb0, tb1,
             sg0, sg1, sg2, sg3, sg4, sw0, sw1):
    rws = (rw0, rw1, rw2, rw3, rw4)
    rps = (rp0, rp1, rp2, rp3, rp4)
    sgs = (sg0, sg1, sg2, sg3, sg4)
    tbs = (tb0, tb1)
    sws = (sw0, sw1)
    wid = lax.axis_index("s") * 2 + lax.axis_index("c")
    col0 = wid * BTILE
    iota = lax.iota(jnp.int32, 16)
    zvec = jnp.zeros((16,), jnp.int32)
    cvecs = [iota + blk * 16 for blk in range(4)]
    fbases = [(iota + blk * 16) * BTILE for blk in range(5)]

    pltpu.sync_copy(wt_hbm.at[:, pl.ds(wid, 1)], iw_all)
    pltpu.sync_copy(pt_hbm.at[:, pl.ds(wid, 1)], ip_all)

    def _idx(all_ref, t):
        return all_ref.at[lax.shift_right_logical(t, 3), 0,
                          lax.bitwise_and(t, 7)]

    def fire_g(s, t):
        pltpu.async_copy(ww_hbm.at[_idx(iw_all, t)], rws[s], sgs[s])
        pltpu.async_copy(wp_hbm.at[_idx(ip_all, t)], rps[s], sgs[s])

    def wait_g(s, t):
        pltpu.make_async_copy(ww_hbm.at[_idx(iw_all, t)],
                              rws[s], sgs[s]).wait()
        pltpu.make_async_copy(wp_hbm.at[_idx(ip_all, t)],
                              rps[s], sgs[s]).wait()

    def wait_w(k, t):
        pltpu.make_async_copy(tbs[k], _out_slab(out_hbm, t, wid),
                              sws[k]).wait()

    def transpose(rw, rp, tb):
        @plsc.parallel_loop(0, BTILE, unroll=8)
        def _(b):
            bb = lax.bitwise_and(iota + b, BTILE - 1)
            for blk in range(5):
                if blk < 4:
                    v = plsc.load_gather(rw, [bb, cvecs[blk]])
                else:
                    v = plsc.load_gather(rp, [bb, iota])
                f = fbases[blk] + bb
                q = lax.shift_right_logical(f, 10)
                r = lax.bitwise_and(f, 1023)
                plsc.store_scatter(tb, [q, zvec, r], v)

    for s in range(NSET):
        fire_g(s, s)

    def body(i, carry):
        t0 = NSET * i
        for s in range(NSET):
            t = t0 + s
            k = s % 2
            wait_g(s, t)
            if s < 2:
                @pl.when(i > 0)
                def _():
                    wait_w(k, t)
            else:
                wait_w(k, t)
            transpose(rws[s], rps[s], tbs[k])
            pltpu.async_copy(tbs[k], _out_slab(out_hbm, t, wid), sws[k])

            @pl.when(i < NITER - 1)
            def _():
                fire_g(s, t + NSET)
        return carry

    lax.fori_loop(0, NITER, body, 0)
    wait_w(0, MAX_LEN - 2)
    wait_w(1, MAX_LEN - 1)


def kernel(word, pos, W_word, W_pos):
    wt = word.T.astype(jnp.int32).reshape(25, 8, NW, BTILE)
    wt = jnp.transpose(wt, (0, 2, 1, 3))
    pt = pos.T.astype(jnp.int32).reshape(25, 8, NW, BTILE)
    pt = jnp.transpose(pt, (0, 2, 1, 3))
    mesh = plsc.VectorSubcoreMesh(core_axis_name="c", subcore_axis_name="s")
    gather_set = [
        pltpu.VMEM((BTILE, WORD_DIM), jnp.float32),
        pltpu.VMEM((BTILE, POS_DIM), jnp.float32),
    ]
    out3 = pl.kernel(
        _sc_body,
        mesh=mesh,
        out_type=jax.ShapeDtypeStruct((MAX_LEN * 10, NW, 1024), jnp.float32),
        compiler_params=pltpu.CompilerParams(
            use_tc_tiling_on_sc=False, needs_layout_passes=False),
        scratch_types=[
            pltpu.VMEM((25, 1, 8, BTILE), jnp.int32),
            pltpu.VMEM((25, 1, 8, BTILE), jnp.int32),
        ] + gather_set * NSET + [
            pltpu.VMEM((10, 1, 1024), jnp.float32),
            pltpu.VMEM((10, 1, 1024), jnp.float32),
        ] + [pltpu.SemaphoreType.DMA] * (NSET + 2),
    )(wt, pt, W_word, W_pos)
    x = out3.reshape(MAX_LEN, 10, NW, 8, BTILE)
    x = jnp.transpose(x, (2, 4, 0, 1, 3))
    return x.reshape(BATCH, MAX_LEN, OUT_DIM)


# speedup vs baseline: 1.0159x; 1.0159x over previous
import jax
import jax.numpy as jnp
from jax import lax
from jax.experimental import pallas as pl
from jax.experimental.pallas import tpu as pltpu
from jax.experimental.pallas import tpu_sc as plsc

BATCH = 4096
MAX_LEN = 200
WORD_DIM = 64
POS_DIM = 16
OUT_DIM = WORD_DIM + POS_DIM
NW = 32
BTILE = BATCH // NW
NSET = 4
NITER = MAX_LEN // NSET


def _out_slab(out_hbm, t, wid):
    return out_hbm.at[pl.ds(t * 10, 10), pl.ds(wid, 1), :]


def _sc_body(wt_hbm, pt_hbm, ww_hbm, wp_hbm, out_hbm,
             iw_all, ip_all,
             rw0, rp0, rw1, rp1, rw2, rp2, rw3, rp3,
             tb0, tb1,
             sg0, sg1, sg2, sg3, sw0, sw1):
    rws = (rw0, rw1, rw2, rw3)
    rps = (rp0, rp1, rp2, rp3)
    sgs = (sg0, sg1, sg2, sg3)
    tbs = (tb0, tb1)
    sws = (sw0, sw1)
    wid = lax.axis_index("s") * 2 + lax.axis_index("c")
    col0 = wid * BTILE
    iota = lax.iota(jnp.int32, 16)
    zvec = jnp.zeros((16,), jnp.int32)
    cvecs = [iota + blk * 16 for blk in range(4)]
    fbases = [(iota + blk * 16) * BTILE for blk in range(5)]

    pltpu.sync_copy(wt_hbm.at[:, pl.ds(wid, 1)], iw_all)
    pltpu.sync_copy(pt_hbm.at[:, pl.ds(wid, 1)], ip_all)

    def _idx(all_ref, t):
        return all_ref.at[lax.shift_right_logical(t, 3), 0,
                          lax.bitwise_and(t, 7)]

    def fire_g(s, t):
        pltpu.async_copy(ww_hbm.at[_idx(iw_all, t)], rws[s], sgs[s])
        pltpu.async_copy(wp_hbm.at[_idx(ip_all, t)], rps[s], sgs[s])

    def wait_g(s, t):
        pltpu.make_async_copy(ww_hbm.at[_idx(iw_all, t)],
                              rws[s], sgs[s]).wait()
        pltpu.make_async_copy(wp_hbm.at[_idx(ip_all, t)],
                              rps[s], sgs[s]).wait()

    def wait_w(k, t):
        pltpu.make_async_copy(tbs[k], _out_slab(out_hbm, t, wid),
                              sws[k]).wait()

    def transpose(rw, rp, tb):
        @plsc.parallel_loop(0, BTILE, unroll=8)
        def _(b):
            bb = lax.bitwise_and(iota + b, BTILE - 1)
            for blk in range(5):
                if blk < 4:
                    v = plsc.load_gather(rw, [bb, cvecs[blk]])
                else:
                    v = plsc.load_gather(rp, [bb, iota])
                f = fbases[blk] + bb
                q = lax.shift_right_logical(f, 10)
                r = lax.bitwise_and(f, 1023)
                plsc.store_scatter(tb, [q, zvec, r], v)

    for s in range(NSET):
        fire_g(s, s)

    def body(i, carry):
        t0 = NSET * i
        for s in range(NSET):
            t = t0 + s
            k = s % 2
            wait_g(s, t)
            if s < 2:
                @pl.when(i > 0)
                def _():
                    wait_w(k, t)
            else:
                wait_w(k, t)
            transpose(rws[s], rps[s], tbs[k])
            pltpu.async_copy(tbs[k], _out_slab(out_hbm, t, wid), sws[k])

            @pl.when(i < NITER - 1)
            def _():
                fire_g(s, t + NSET)
        return carry

    lax.fori_loop(0, NITER, body, 0)
    wait_w(0, MAX_LEN - 2)
    wait_w(1, MAX_LEN - 1)


def kernel(word, pos, W_word, W_pos):
    wt = word.T.astype(jnp.int32).reshape(25, 8, NW, BTILE)
    wt = jnp.transpose(wt, (0, 2, 1, 3))
    pt = pos.T.astype(jnp.int32).reshape(25, 8, NW, BTILE)
    pt = jnp.transpose(pt, (0, 2, 1, 3))
    mesh = plsc.VectorSubcoreMesh(core_axis_name="c", subcore_axis_name="s")
    gather_set = [
        pltpu.VMEM((BTILE, WORD_DIM), jnp.float32),
        pltpu.VMEM((BTILE, POS_DIM), jnp.float32),
    ]
    out3 = pl.kernel(
        _sc_body,
        mesh=mesh,
        out_type=jax.ShapeDtypeStruct((MAX_LEN * 10, NW, 1024), jnp.float32),
        compiler_params=pltpu.CompilerParams(
            use_tc_tiling_on_sc=False, needs_layout_passes=False),
        scratch_types=[
            pltpu.VMEM((25, 1, 8, BTILE), jnp.int32),
            pltpu.VMEM((25, 1, 8, BTILE), jnp.int32),
        ] + gather_set * NSET + [
            pltpu.VMEM((10, 1, 1024), jnp.float32),
            pltpu.VMEM((10, 1, 1024), jnp.float32),
        ] + [pltpu.SemaphoreType.DMA] * (NSET + 2),
    )(wt, pt, W_word, W_pos)
    x = out3.reshape(MAX_LEN, 10, NW, 8, BTILE)
    x = jnp.transpose(x, (2, 4, 0, 1, 3))
    return x.reshape(BATCH, MAX_LEN, OUT_DIM)
